# baseline (speedup 1.0000x reference)
import functools

import jax
import jax.numpy as jnp
from jax import lax
from jax.experimental import pallas as pl
from jax.experimental.pallas import tpu as pltpu
from jax.experimental.pallas import tpu_sc as plsc

_D = 64
_DP = 128
_NC, _NS = 2, 16
_NW = _NC * _NS
_H = 50
_HP = 56
_K = 64
_NBUF = 4
_AHEAD = 2


@functools.cache
def _make_compact(BATCH: int):
    t_per_w = BATCH // 8 // _NW
    rows_w = t_per_w * 8
    n_c = rows_w * _HP // _K
    mesh = plsc.VectorSubcoreMesh(core_axis_name="c", subcore_axis_name="s")

    @functools.partial(
        pl.kernel,
        mesh=mesh,
        compiler_params=pltpu.CompilerParams(needs_layout_passes=False),
        out_type=jax.ShapeDtypeStruct((BATCH * _HP // _K, _K), jnp.int32),
        scratch_types=[
            pltpu.VMEM((t_per_w, 8, _H), jnp.int32),
            pltpu.VMEM((n_c, _K), jnp.int32),
        ],
    )
    def compact_kernel(x_hbm, idx_hbm, xs, iout):
        wid = lax.axis_index("s") * _NC + lax.axis_index("c")
        pltpu.sync_copy(x_hbm.at[pl.ds(wid * t_per_w, t_per_w)], xs)
        lanes = lax.iota(jnp.int32, 16)

        def put(slot0, v, mask=None):
            slot = lanes + slot0
            plsc.store_scatter(iout, [slot // _K, slot % _K], v, mask=mask)

        def tile_body(tl, _):
            for j in range(8):
                off = (tl * 8 + j) * _HP
                for k in range(3):
                    put(off + 16 * k, xs[tl, j, pl.ds(16 * k, 16)])
                put(off + 34, xs[tl, j, pl.ds(34, 16)], mask=lanes >= 14)
                put(off + 50, lanes + (off + 50), mask=lanes < 6)
            return 0

        lax.fori_loop(0, t_per_w, tile_body, 0)
        pltpu.sync_copy(iout, idx_hbm.at[pl.ds(wid * n_c, n_c)])

    return compact_kernel


@functools.cache
def _make_gather(BATCH: int):
    n_slots = BATCH * _HP
    n_chunks = n_slots // _K // _NW
    rounds = n_chunks // _NBUF
    assert n_chunks % _NBUF == 0 and rounds >= 3
    mesh = plsc.VectorSubcoreMesh(core_axis_name="c", subcore_axis_name="s")

    @functools.partial(
        pl.kernel,
        mesh=mesh,
        compiler_params=pltpu.CompilerParams(use_tc_tiling_on_sc=False),
        out_type=jax.ShapeDtypeStruct((n_slots, _DP), jnp.float32),
        scratch_types=(
            [pltpu.VMEM((n_chunks, _K), jnp.int32)]
            + [pltpu.VMEM((_K, _D), jnp.float32)] * _NBUF
            + [pltpu.VMEM((_K, _DP), jnp.float32)] * _NBUF
            + [pltpu.SemaphoreType.DMA] * (2 * _NBUF)
        ),
    )
    def gather_kernel(idx_hbm, table_hbm, out_hbm, idx_v, *bufs):
        gb = bufs[:_NBUF]
        wb = bufs[_NBUF:2 * _NBUF]
        gsem = bufs[2 * _NBUF:3 * _NBUF]
        wsem = bufs[3 * _NBUF:]
        wid = lax.axis_index("s") * _NC + lax.axis_index("c")
        base = wid * n_chunks * _K

        pltpu.sync_copy(idx_hbm.at[pl.ds(wid * n_chunks, n_chunks)], idx_v)

        def start_gather(c, b):
            pltpu.async_copy(table_hbm.at[idx_v.at[c]], gb[b], gsem[b])

        def wait_gather(c, b):
            pltpu.make_async_copy(table_hbm.at[idx_v.at[c]], gb[b],
                                  gsem[b]).wait()

        def out_slice(c):
            return out_hbm.at[pl.ds(base + c * _K, _K)]

        def start_write(c, b):
            pltpu.async_copy(wb[b], out_slice(c), wsem[b])

        def wait_write(c, b):
            pltpu.make_async_copy(wb[b], out_slice(c), wsem[b]).wait()

        def expand(b):
            g, w = gb[b], wb[b]

            def ex4(i, _):
                r0 = i * 4
                for rr in range(4):
                    for k in range(4):
                        w[r0 + rr, pl.ds(16 * k, 16)] = (
                            g[r0 + rr, pl.ds(16 * k, 16)])
                return 0

            lax.fori_loop(0, _K // 4, ex4, 0)

        def step(c, b, do_wait_prev_write, do_next_gather):
            wait_gather(c, b)
            expand(b)
            start_write(c, b)
            if do_next_gather:
                bn = (b + _AHEAD) % _NBUF
                if do_wait_prev_write:
                    wait_write(c + _AHEAD - _NBUF, bn)
                start_gather(c + _AHEAD, bn)

        for b in range(_AHEAD):
            start_gather(b, b)

        for b in range(_NBUF):
            step(b, b, do_wait_prev_write=(b + _AHEAD >= _NBUF),
                 do_next_gather=True)

        def mid_round(r, _):
            for b in range(_NBUF):
                step(r * _NBUF + b, b, True, True)
            return 0

        lax.fori_loop(1, rounds - 1, mid_round, 0)

        c_last = (rounds - 1) * _NBUF
        for b in range(_NBUF):
            step(c_last + b, b, do_wait_prev_write=(b < _AHEAD),
                 do_next_gather=(b < _AHEAD))

        for b in range(_NBUF):
            wait_write(n_chunks - _NBUF + b, b)

    return gather_kernel


def kernel(x, table):
    batch, h = x.shape
    assert h == _H
    xv = x.astype(jnp.int32).reshape(batch // 8, 8, _H)
    idx56 = _make_compact(batch)(xv)
    out = _make_gather(batch)(idx56, table)
    return out.reshape(batch, _HP, _DP)[:, :_H, :_D]


# speedup vs baseline: 1.1560x; 1.1560x over previous
import functools

import jax
import jax.numpy as jnp
from jax import lax
from jax.experimental import pallas as pl
from jax.experimental.pallas import tpu as pltpu
from jax.experimental.pallas import tpu_sc as plsc

_D = 64
_DP = 128
_NC, _NS = 2, 16
_NW = _NC * _NS
_H = 50
_HP = 56
_K = 128
_NBUF = 4
_AHEAD = 2


@functools.cache
def _make_compact(BATCH: int):
    t_per_w = BATCH // 8 // _NW
    rows_w = t_per_w * 8
    n_c = rows_w * _HP // _K
    mesh = plsc.VectorSubcoreMesh(core_axis_name="c", subcore_axis_name="s")

    @functools.partial(
        pl.kernel,
        mesh=mesh,
        compiler_params=pltpu.CompilerParams(needs_layout_passes=False),
        out_type=jax.ShapeDtypeStruct((BATCH * _HP // _K, _K), jnp.int32),
        scratch_types=[
            pltpu.VMEM((t_per_w, 8, _H), jnp.int32),
            pltpu.VMEM((n_c, _K), jnp.int32),
        ],
    )
    def compact_kernel(x_hbm, idx_hbm, xs, iout):
        wid = lax.axis_index("s") * _NC + lax.axis_index("c")
        pltpu.sync_copy(x_hbm.at[pl.ds(wid * t_per_w, t_per_w)], xs)
        lanes = lax.iota(jnp.int32, 16)

        def put(slot0, v, mask=None):
            slot = lanes + slot0
            plsc.store_scatter(iout, [slot // _K, slot % _K], v, mask=mask)

        def tile_body(tl, _):
            for j in range(8):
                off = (tl * 8 + j) * _HP
                for k in range(3):
                    put(off + 16 * k, xs[tl, j, pl.ds(16 * k, 16)])
                put(off + 34, xs[tl, j, pl.ds(34, 16)], mask=lanes >= 14)
                put(off + 50, lanes + (off + 50), mask=lanes < 6)
            return 0

        lax.fori_loop(0, t_per_w, tile_body, 0)
        pltpu.sync_copy(iout, idx_hbm.at[pl.ds(wid * n_c, n_c)])

    return compact_kernel


@functools.cache
def _make_gather(BATCH: int):
    n_slots = BATCH * _HP
    n_chunks = n_slots // _K // _NW
    rounds = n_chunks // _NBUF
    assert n_chunks % _NBUF == 0 and rounds >= 3
    mesh = plsc.VectorSubcoreMesh(core_axis_name="c", subcore_axis_name="s")

    @functools.partial(
        pl.kernel,
        mesh=mesh,
        compiler_params=pltpu.CompilerParams(use_tc_tiling_on_sc=False),
        out_type=jax.ShapeDtypeStruct((n_slots, _DP), jnp.float32),
        scratch_types=(
            [pltpu.VMEM((n_chunks, _K), jnp.int32)]
            + [pltpu.VMEM((_K, _DP), jnp.float32)] * _NBUF
            + [pltpu.SemaphoreType.DMA] * (2 * _NBUF)
        ),
    )
    def gather_kernel(idx_hbm, table_hbm, out_hbm, idx_v, *bufs):
        rows = bufs[:_NBUF]
        gsem = bufs[_NBUF:2 * _NBUF]
        wsem = bufs[2 * _NBUF:]
        wid = lax.axis_index("s") * _NC + lax.axis_index("c")
        base = wid * n_chunks * _K

        pltpu.sync_copy(idx_hbm.at[pl.ds(wid * n_chunks, n_chunks)], idx_v)

        def start_gather(c, b):
            pltpu.async_copy(table_hbm.at[idx_v.at[c]], rows[b], gsem[b])

        def wait_gather(c, b):
            pltpu.make_async_copy(table_hbm.at[idx_v.at[c]], rows[b],
                                  gsem[b]).wait()

        def out_slice(c):
            return out_hbm.at[pl.ds(base + c * _K, _K)]

        def start_write(c, b):
            pltpu.async_copy(rows[b], out_slice(c), wsem[b])

        def wait_write(c, b):
            pltpu.make_async_copy(rows[b], out_slice(c), wsem[b]).wait()

        def step(c, b, do_wait_prev_write, do_next_gather):
            wait_gather(c, b)
            start_write(c, b)
            if do_next_gather:
                bn = (b + _AHEAD) % _NBUF
                if do_wait_prev_write:
                    wait_write(c + _AHEAD - _NBUF, bn)
                start_gather(c + _AHEAD, bn)

        for b in range(_AHEAD):
            start_gather(b, b)

        for b in range(_NBUF):
            step(b, b, do_wait_prev_write=(b + _AHEAD >= _NBUF),
                 do_next_gather=True)

        def mid_round(r, _):
            for b in range(_NBUF):
                step(r * _NBUF + b, b, True, True)
            return 0

        lax.fori_loop(1, rounds - 1, mid_round, 0)

        c_last = (rounds - 1) * _NBUF
        for b in range(_NBUF):
            step(c_last + b, b, do_wait_prev_write=(b < _AHEAD),
                 do_next_gather=(b < _AHEAD))

        for b in range(_NBUF):
            wait_write(n_chunks - _NBUF + b, b)

    return gather_kernel


def kernel(x, table):
    batch, h = x.shape
    assert h == _H
    xv = x.astype(jnp.int32).reshape(batch // 8, 8, _H)
    idx56 = _make_compact(batch)(xv)
    tpad = jnp.pad(table, ((0, 0), (0, _DP - _D)))
    out = _make_gather(batch)(idx56, tpad)
    return out.reshape(batch, _HP, _DP)[:, :_H, :_D]
